# baseline (speedup 1.0000x reference)
import functools

import jax
import jax.numpy as jnp
from jax import lax
from jax.experimental import pallas as pl
from jax.experimental.pallas import tpu as pltpu
from jax.experimental.pallas import tpu_sc as plsc

_NC = 2
_NS = 16
_NW = _NC * _NS


def _sc_body(pairs_per_w, a, b, o1, o2, sem):
    wid = lax.axis_index("s") * _NC + lax.axis_index("c")
    base = wid * pairs_per_w
    copies = []
    for j in range(pairs_per_w):
        r = base + j
        copies.append(pltpu.async_copy(b.at[r, 0], o1.at[r, 0], sem))
        copies.append(pltpu.async_copy(a.at[r, 1], o1.at[r, 1], sem))
        copies.append(pltpu.async_copy(a.at[r, 0], o2.at[r, 0], sem))
        copies.append(pltpu.async_copy(b.at[r, 1], o2.at[r, 1], sem))
    for c in copies:
        c.wait()


def kernel(x1, x2):
    N, C, H, W = x1.shape
    R = N * C // 2
    pairs_per_w = R // _NW
    a = x1.reshape(R, 2, H, W)
    b = x2.reshape(R, 2, H, W)
    mesh = plsc.VectorSubcoreMesh(core_axis_name="c", subcore_axis_name="s")
    run = pl.kernel(
        functools.partial(_sc_body, pairs_per_w),
        out_type=[jax.ShapeDtypeStruct((R, 2, H, W), jnp.float32)] * 2,
        mesh=mesh,
        scratch_types=[pltpu.SemaphoreType.DMA],
    )
    o1, o2 = run(a, b)
    return o1.reshape(N, C, H, W), o2.reshape(N, C, H, W)


# speedup vs baseline: 39.0728x; 39.0728x over previous
import functools

import jax
import jax.numpy as jnp
from jax import lax
from jax.experimental import pallas as pl
from jax.experimental.pallas import tpu as pltpu
from jax.experimental.pallas import tpu_sc as plsc

_NC = 2
_NS = 16
_NW = _NC * _NS


def _sc_body(pairs_per_w, a, b, o1, o2, buf, sem_in, sem_out):
    wid = lax.axis_index("s") * _NC + lax.axis_index("c")
    base = wid * pairs_per_w
    jobs = []
    for j in range(pairs_per_w):
        r = base + j
        jobs.append((b, o1, r, 0))
        jobs.append((a, o1, r, 1))
        jobs.append((a, o2, r, 0))
        jobs.append((b, o2, r, 1))
    nj = len(jobs)

    def start_in(i, slot):
        src = jobs[i][0]
        return pltpu.async_copy(src.at[jobs[i][2], jobs[i][3]],
                                buf.at[slot], sem_in)

    def start_out(i, slot):
        dst = jobs[i][1]
        return pltpu.async_copy(buf.at[slot],
                                dst.at[jobs[i][2], jobs[i][3]], sem_out)

    ins = [None, None]
    outs = [None, None]
    ins[0] = start_in(0, 0)
    for i in range(nj):
        slot = i % 2
        nslot = (i + 1) % 2
        if i + 1 < nj:
            if outs[nslot] is not None:
                outs[nslot].wait()
            ins[nslot] = start_in(i + 1, nslot)
        ins[slot].wait()
        outs[slot] = start_out(i, slot)
    for o in outs:
        if o is not None:
            o.wait()


def kernel(x1, x2):
    N, C, H, W = x1.shape
    R = N * C // 2
    pairs_per_w = R // _NW
    a = x1.reshape(R, 2, H, W)
    b = x2.reshape(R, 2, H, W)
    mesh = plsc.VectorSubcoreMesh(core_axis_name="c", subcore_axis_name="s")
    run = pl.kernel(
        functools.partial(_sc_body, pairs_per_w),
        out_type=[jax.ShapeDtypeStruct((R, 2, H, W), jnp.float32)] * 2,
        mesh=mesh,
        scratch_types=[
            pltpu.VMEM((2, H, W), jnp.float32),
            pltpu.SemaphoreType.DMA,
            pltpu.SemaphoreType.DMA,
        ],
    )
    o1, o2 = run(a, b)
    return o1.reshape(N, C, H, W), o2.reshape(N, C, H, W)
